# baseline (speedup 1.0000x reference)
import jax
import jax.numpy as jnp
from jax import lax
from jax.experimental import pallas as pl
from jax.experimental.pallas import tpu as pltpu
from jax.experimental.pallas import tpu_sc as plsc

DIM = 32
NROW = 4096
NCOL = 200

_info = plsc.get_sparse_core_info()
NC = _info.num_cores
NS = _info.num_subcores
NW = NC * NS

RT = NCOL // 8
CT = NROW // 128
ST = DIM // 8


def _gather_body(idxp_hbm, table_hbm, out_hbm, idx_v, rows_v, outt_v, gsem):
    ct = lax.axis_index("s") * NC + lax.axis_index("c")

    pltpu.sync_copy(idxp_hbm.at[:, ct], idx_v)

    lane = lax.iota(jnp.int32, 16)

    def tile(t, carry):
        rt = t // 8
        sr = t % 8
        pltpu.make_async_copy(
            table_hbm.at[idx_v.at[rt, sr]], rows_v, gsem
        ).start()
        pltpu.make_async_copy(
            table_hbm.at[idx_v.at[0, 0]], rows_v, gsem
        ).wait()
        for i in range(128):
            for h in range(2):
                x = rows_v[i, pl.ds(h * 16, 16)]
                dest = (lane + h * 16) * 128 + i
                plsc.store_scatter(outt_v, [dest], x)
        for st in range(ST):
            pltpu.sync_copy(
                outt_v.at[pl.ds(st * 1024, 1024)], out_hbm.at[t, st, ct]
            )
        return carry

    lax.fori_loop(0, NCOL, tile, 0)


_mesh = plsc.VectorSubcoreMesh(core_axis_name="c", subcore_axis_name="s")

_gather = pl.kernel(
    _gather_body,
    out_type=jax.ShapeDtypeStruct((NCOL, ST, CT, 1024), jnp.float32),
    mesh=_mesh,
    scratch_types=[
        pltpu.VMEM((RT, 8, 128), jnp.int32),
        pltpu.VMEM((128, DIM), jnp.float32),
        pltpu.VMEM((DIM * 128,), jnp.float32),
        pltpu.SemaphoreType.DMA,
    ],
    compiler_params=pltpu.CompilerParams(
        use_tc_tiling_on_sc=False, needs_layout_passes=False
    ),
)


@jax.jit
def kernel(data, table):
    dataP = data.T.reshape(RT, 8, CT, 128).transpose(0, 2, 1, 3)
    o4 = _gather(dataP.astype(jnp.int32), table)
    o5 = o4.reshape(NCOL, ST, CT, 8, 128)
    out = o5.transpose(2, 4, 0, 1, 3).reshape(NROW, NCOL, DIM)
    return out


# speedup vs baseline: 1.2298x; 1.2298x over previous
import jax
import jax.numpy as jnp
from jax import lax
from jax.experimental import pallas as pl
from jax.experimental.pallas import tpu as pltpu
from jax.experimental.pallas import tpu_sc as plsc

DIM = 32
NROW = 4096
NCOL = 200

_info = plsc.get_sparse_core_info()
NC = _info.num_cores
NS = _info.num_subcores
NW = NC * NS

RT = NCOL // 8
CT = NROW // 128
ST = DIM // 8


def _gather_body(idxp_hbm, table_hbm, out_hbm,
                 idx_v, rows0, rows1, outt0, outt1,
                 gsem0, gsem1, ssem0, ssem1):
    ct = lax.axis_index("s") * NC + lax.axis_index("c")
    rows = (rows0, rows1)
    outt = (outt0, outt1)
    gsem = (gsem0, gsem1)
    ssem = (ssem0, ssem1)

    pltpu.sync_copy(idxp_hbm.at[:, ct], idx_v)

    lane = lax.iota(jnp.int32, 16)

    def start_gather(t, b):
        pltpu.make_async_copy(
            table_hbm.at[idx_v.at[t // 8, t % 8]], rows[b], gsem[b]
        ).start()

    def start_gather_dyn(t, b):
        rt = t // 8
        sr = t % 8
        pltpu.make_async_copy(
            table_hbm.at[idx_v.at[rt, sr]], rows[b], gsem[b]
        ).start()

    def wait_gather(b):
        pltpu.make_async_copy(
            table_hbm.at[idx_v.at[0, 0]], rows[b], gsem[b]
        ).wait()

    def start_store(t, o):
        for st in range(ST):
            pltpu.make_async_copy(
                outt[o].at[pl.ds(st * 1024, 1024)], out_hbm.at[t, st, ct],
                ssem[o],
            ).start()

    def wait_store(o):
        for st in range(ST):
            pltpu.make_async_copy(
                outt[o].at[pl.ds(st * 1024, 1024)], out_hbm.at[0, st, ct],
                ssem[o],
            ).wait()

    def transpose(b, o):
        for i in range(128):
            for h in range(2):
                x = rows[b][i, pl.ds(h * 16, 16)]
                dest = (lane + h * 16) * 128 + i
                plsc.store_scatter(outt[o], [dest], x)

    start_gather(0, 0)

    def pair(i, carry):
        t0 = 2 * i
        wait_gather(0)
        start_gather_dyn(t0 + 1, 1)

        @pl.when(i > 0)
        def _():
            wait_store(0)

        transpose(0, 0)
        start_store(t0, 0)

        wait_gather(1)
        start_gather_dyn(lax.rem(t0 + 2, NCOL), 0)

        @pl.when(i > 0)
        def _():
            wait_store(1)

        transpose(1, 1)
        start_store(t0 + 1, 1)
        return carry

    lax.fori_loop(0, NCOL // 2, pair, 0)

    wait_gather(0)
    wait_store(0)
    wait_store(1)


_mesh = plsc.VectorSubcoreMesh(core_axis_name="c", subcore_axis_name="s")

_gather = pl.kernel(
    _gather_body,
    out_type=jax.ShapeDtypeStruct((NCOL, ST, CT, 1024), jnp.float32),
    mesh=_mesh,
    scratch_types=[
        pltpu.VMEM((RT, 8, 128), jnp.int32),
        pltpu.VMEM((128, DIM), jnp.float32),
        pltpu.VMEM((128, DIM), jnp.float32),
        pltpu.VMEM((DIM * 128,), jnp.float32),
        pltpu.VMEM((DIM * 128,), jnp.float32),
        pltpu.SemaphoreType.DMA,
        pltpu.SemaphoreType.DMA,
        pltpu.SemaphoreType.DMA,
        pltpu.SemaphoreType.DMA,
    ],
    compiler_params=pltpu.CompilerParams(
        use_tc_tiling_on_sc=False, needs_layout_passes=False
    ),
)


@jax.jit
def kernel(data, table):
    dataP = data.T.reshape(RT, 8, CT, 128).transpose(0, 2, 1, 3)
    o4 = _gather(dataP.astype(jnp.int32), table)
    o5 = o4.reshape(NCOL, ST, CT, 8, 128)
    out = o5.transpose(2, 4, 0, 1, 3).reshape(NROW, NCOL, DIM)
    return out


# speedup vs baseline: 1.3066x; 1.0625x over previous
import jax
import jax.numpy as jnp
from jax import lax
from jax.experimental import pallas as pl
from jax.experimental.pallas import tpu as pltpu
from jax.experimental.pallas import tpu_sc as plsc

DIM = 32
NROW = 4096
NCOL = 200

_info = plsc.get_sparse_core_info()
NC = _info.num_cores
NS = _info.num_subcores
NW = NC * NS

RT = NCOL // 8
CT = NROW // 128
ST = DIM // 8


def _gather_body(idxp_hbm, table_hbm, out_hbm,
                 idx_v, rows0, rows1, outt0, outt1,
                 gsem0, gsem1, ssem0, ssem1):
    ct = lax.axis_index("s") * NC + lax.axis_index("c")
    rows = (rows0, rows1)
    outt = (outt0, outt1)
    gsem = (gsem0, gsem1)
    ssem = (ssem0, ssem1)

    pltpu.sync_copy(idxp_hbm.at[:, ct], idx_v)

    lane = lax.iota(jnp.int32, 16)

    def start_gather(t, b):
        pltpu.make_async_copy(
            table_hbm.at[idx_v.at[t // 8, t % 8]], rows[b], gsem[b]
        ).start()

    def start_gather_dyn(t, b):
        rt = t // 8
        sr = t % 8
        pltpu.make_async_copy(
            table_hbm.at[idx_v.at[rt, sr]], rows[b], gsem[b]
        ).start()

    def wait_gather(b):
        pltpu.make_async_copy(
            table_hbm.at[idx_v.at[0, 0]], rows[b], gsem[b]
        ).wait()

    def start_store(t, o):
        for st in range(ST):
            pltpu.make_async_copy(
                outt[o].at[pl.ds(st * 1024, 1024)], out_hbm.at[t, st, ct],
                ssem[o],
            ).start()

    def wait_store(o):
        for st in range(ST):
            pltpu.make_async_copy(
                outt[o].at[pl.ds(st * 1024, 1024)], out_hbm.at[0, st, ct],
                ssem[o],
            ).wait()

    base = (lane * 128, lane * 128 + 16 * 128)

    def transpose(b, o):
        for i0 in range(0, 128, 8):
            xs = [rows[b][i0 + k, pl.ds(h * 16, 16)]
                  for k in range(8) for h in range(2)]
            for k in range(8):
                for h in range(2):
                    plsc.store_scatter(
                        outt[o], [base[h] + (i0 + k)], xs[k * 2 + h]
                    )

    start_gather(0, 0)

    def pair(i, carry):
        t0 = 2 * i
        wait_gather(0)
        start_gather_dyn(t0 + 1, 1)

        @pl.when(i > 0)
        def _():
            wait_store(0)

        transpose(0, 0)
        start_store(t0, 0)

        wait_gather(1)
        start_gather_dyn(lax.rem(t0 + 2, NCOL), 0)

        @pl.when(i > 0)
        def _():
            wait_store(1)

        transpose(1, 1)
        start_store(t0 + 1, 1)
        return carry

    lax.fori_loop(0, NCOL // 2, pair, 0)

    wait_gather(0)
    wait_store(0)
    wait_store(1)


_mesh = plsc.VectorSubcoreMesh(core_axis_name="c", subcore_axis_name="s")

_gather = pl.kernel(
    _gather_body,
    out_type=jax.ShapeDtypeStruct((NCOL, ST, CT, 1024), jnp.float32),
    mesh=_mesh,
    scratch_types=[
        pltpu.VMEM((RT, 8, 128), jnp.int32),
        pltpu.VMEM((128, DIM), jnp.float32),
        pltpu.VMEM((128, DIM), jnp.float32),
        pltpu.VMEM((DIM * 128,), jnp.float32),
        pltpu.VMEM((DIM * 128,), jnp.float32),
        pltpu.SemaphoreType.DMA,
        pltpu.SemaphoreType.DMA,
        pltpu.SemaphoreType.DMA,
        pltpu.SemaphoreType.DMA,
    ],
    compiler_params=pltpu.CompilerParams(
        use_tc_tiling_on_sc=False, needs_layout_passes=False
    ),
)


@jax.jit
def kernel(data, table):
    dataP = data.T.reshape(RT, 8, CT, 128).transpose(0, 2, 1, 3)
    o4 = _gather(dataP.astype(jnp.int32), table)
    o5 = o4.reshape(NCOL, ST, CT, 8, 128)
    out = o5.transpose(2, 4, 0, 1, 3).reshape(NROW, NCOL, DIM)
    return out


# speedup vs baseline: 1.3081x; 1.0011x over previous
import jax
import jax.numpy as jnp
from jax import lax
from jax.experimental import pallas as pl
from jax.experimental.pallas import tpu as pltpu
from jax.experimental.pallas import tpu_sc as plsc

DIM = 32
NROW = 4096
NCOL = 200

_info = plsc.get_sparse_core_info()
NC = _info.num_cores
NS = _info.num_subcores
NW = NC * NS

RT = NCOL // 8
CT = NROW // 128
ST = DIM // 8


def _gather_body(idxp_hbm, table_hbm, out_hbm,
                 idx_v, rows0, rows1, outt0, outt1,
                 gsem0, gsem1, ssem0, ssem1):
    ct = lax.axis_index("s") * NC + lax.axis_index("c")
    rows = (rows0, rows1)
    outt = (outt0, outt1)
    gsem = (gsem0, gsem1)
    ssem = (ssem0, ssem1)

    pltpu.sync_copy(idxp_hbm.at[:, ct], idx_v)

    lane = lax.iota(jnp.int32, 16)

    def start_gather(t, b):
        pltpu.make_async_copy(
            table_hbm.at[idx_v.at[t // 8, t % 8]], rows[b], gsem[b]
        ).start()

    def start_gather_dyn(t, b):
        rt = t // 8
        sr = t % 8
        pltpu.make_async_copy(
            table_hbm.at[idx_v.at[rt, sr]], rows[b], gsem[b]
        ).start()

    def wait_gather(b):
        pltpu.make_async_copy(
            table_hbm.at[idx_v.at[0, 0]], rows[b], gsem[b]
        ).wait()

    def start_store(t, o):
        for st in range(ST):
            pltpu.make_async_copy(
                outt[o].at[pl.ds(st * 1024, 1024)], out_hbm.at[t, st, ct],
                ssem[o],
            ).start()

    def wait_store(o):
        for st in range(ST):
            pltpu.make_async_copy(
                outt[o].at[pl.ds(st * 1024, 1024)], out_hbm.at[0, st, ct],
                ssem[o],
            ).wait()

    base = (lane * 128, lane * 128 + 16 * 128)
    G = 8
    SLC = DIM * 128 - 128 + G

    def transpose(b, o):
        def loads(i0):
            return [rows[b][i0 + k, pl.ds(h * 16, 16)]
                    for k in range(G) for h in range(2)]

        def stores(i0, xs):
            for k in range(G):
                for h in range(2):
                    plsc.store_scatter(
                        outt[o].at[pl.ds(i0, SLC)], [base[h] + k],
                        xs[k * 2 + h],
                    )

        xs = loads(0)
        for i0 in range(G, 128, G):
            ys = loads(i0)
            stores(i0 - G, xs)
            xs = ys
        stores(128 - G, xs)

    start_gather(0, 0)

    def pair(i, carry):
        t0 = 2 * i
        wait_gather(0)
        start_gather_dyn(t0 + 1, 1)

        @pl.when(i > 0)
        def _():
            wait_store(0)

        transpose(0, 0)
        start_store(t0, 0)

        wait_gather(1)
        start_gather_dyn(lax.rem(t0 + 2, NCOL), 0)

        @pl.when(i > 0)
        def _():
            wait_store(1)

        transpose(1, 1)
        start_store(t0 + 1, 1)
        return carry

    lax.fori_loop(0, NCOL // 2, pair, 0)

    wait_gather(0)
    wait_store(0)
    wait_store(1)


_mesh = plsc.VectorSubcoreMesh(core_axis_name="c", subcore_axis_name="s")

_gather = pl.kernel(
    _gather_body,
    out_type=jax.ShapeDtypeStruct((NCOL, ST, CT, 1024), jnp.float32),
    mesh=_mesh,
    scratch_types=[
        pltpu.VMEM((RT, 8, 128), jnp.int32),
        pltpu.VMEM((128, DIM), jnp.float32),
        pltpu.VMEM((128, DIM), jnp.float32),
        pltpu.VMEM((DIM * 128,), jnp.float32),
        pltpu.VMEM((DIM * 128,), jnp.float32),
        pltpu.SemaphoreType.DMA,
        pltpu.SemaphoreType.DMA,
        pltpu.SemaphoreType.DMA,
        pltpu.SemaphoreType.DMA,
    ],
    compiler_params=pltpu.CompilerParams(
        use_tc_tiling_on_sc=False, needs_layout_passes=False
    ),
)


@jax.jit
def kernel(data, table):
    dataP = data.T.reshape(RT, 8, CT, 128).transpose(0, 2, 1, 3)
    o4 = _gather(dataP.astype(jnp.int32), table)
    o5 = o4.reshape(NCOL, ST, CT, 8, 128)
    out = o5.transpose(2, 4, 0, 1, 3).reshape(NROW, NCOL, DIM)
    return out
